# Initial kernel scaffold; baseline (speedup 1.0000x reference)
import jax
import jax.numpy as jnp
from jax.experimental import pallas as pl


def kernel(x, adj, W1, b1, W2, b2):
    raise NotImplementedError("write your pallas kernel here")


# speedup vs baseline: 9.7861x; 9.7861x over previous
import dataclasses
import functools

import jax
import jax.numpy as jnp
from jax import lax
from jax.experimental import pallas as pl
from jax.experimental.pallas import tpu as pltpu
from jax.experimental.pallas import tpu_sc as plsc

N = 10000
E = 320000
NC = 2
NS = 16
NW = NC * NS
ET = E // NW
C = 80
NCHUNK = ET // C
L = 16
NP = 10112
SP = NP // NS

_mesh = plsc.VectorSubcoreMesh(core_axis_name="c", subcore_axis_name="s")

_sc_params = pltpu.CompilerParams()
if "needs_layout_passes" in pltpu.CompilerParams.__dataclass_fields__:
    _sc_params = dataclasses.replace(
        _sc_params, needs_layout_passes=False, use_tc_tiling_on_sc=False)


def _zero_vmem(ref, rows, cols):
    @pl.loop(0, rows)
    def _(r):
        for k in range(cols // L):
            ref[r, pl.ds(k * L, L)] = jnp.zeros((L,), jnp.float32)


def _att_pass(xn, src, dst, D):
    @functools.partial(
        pl.kernel,
        out_type=(
            jax.ShapeDtypeStruct((E,), jnp.float32),
            jax.ShapeDtypeStruct((NC, NP, L), jnp.float32),
        ),
        mesh=_mesh,
        compiler_params=_sc_params,
        scratch_types=[
            pltpu.VMEM((C,), jnp.int32),
            pltpu.VMEM((C,), jnp.int32),
            pltpu.VMEM((C, D), jnp.float32),
            pltpu.VMEM((C, D), jnp.float32),
            pltpu.VMEM((C,), jnp.float32),
            pltpu.VMEM((C, L), jnp.float32),
            pltpu.VMEM((SP, L), jnp.float32),
            pltpu.VMEM_SHARED((NP, L), jnp.float32),
            pltpu.SemaphoreType.DMA,
            pltpu.SemaphoreType.DMA,
        ],
    )
    def att(xn_hbm, src_hbm, dst_hbm, ew_hbm, degp_hbm,
            idx_s, idx_d, a_buf, b_buf, ew_buf, drow, zbuf, deg_sh,
            sem_a, sem_b):
        cid = lax.axis_index("c")
        sid = lax.axis_index("s")
        wid = sid * NC + cid

        _zero_vmem(zbuf, SP, L)
        _zero_vmem(drow, C, L)
        pltpu.sync_copy(zbuf, deg_sh.at[pl.ds(sid * SP, SP)])
        plsc.subcore_barrier()

        lane_iota = lax.iota(jnp.int32, L)
        zeros_i = jnp.zeros((L,), jnp.int32)

        @pl.loop(0, NCHUNK)
        def _(i):
            base = wid * ET + i * C
            pltpu.sync_copy(src_hbm.at[pl.ds(base, C)], idx_s)
            pltpu.sync_copy(dst_hbm.at[pl.ds(base, C)], idx_d)
            cp_a = pltpu.async_copy(xn_hbm.at[idx_s], a_buf, sem_a)
            cp_b = pltpu.async_copy(xn_hbm.at[idx_d], b_buf, sem_b)
            cp_a.wait()
            cp_b.wait()

            @pl.loop(0, C // L)
            def _(g):
                sims = jnp.zeros((L,), jnp.float32)
                for e in range(L):
                    row = g * L + e
                    acc = a_buf[row, pl.ds(0, L)] * b_buf[row, pl.ds(0, L)]
                    for k in range(1, D // L):
                        acc += (a_buf[row, pl.ds(k * L, L)]
                                * b_buf[row, pl.ds(k * L, L)])
                    sims = jnp.where(lane_iota == e,
                                     jnp.broadcast_to(jnp.sum(acc), (L,)),
                                     sims)
                sims = jnp.where(sims < 0.1, 0.0, sims)
                ew_buf[pl.ds(g * L, L)] = sims
                plsc.store_scatter(drow, [g * L + lane_iota, zeros_i], sims)

            pltpu.sync_copy(ew_buf, ew_hbm.at[pl.ds(base, C)])
            pltpu.sync_copy(drow, deg_sh.at[idx_d], add=True)

        plsc.subcore_barrier()
        pltpu.sync_copy(deg_sh.at[pl.ds(sid * SP, SP)],
                        degp_hbm.at[cid, pl.ds(sid * SP, SP)])

    return att(xn, src, dst)


def _msg_pass(h, src, dst, ew, dinv, Dm):
    @functools.partial(
        pl.kernel,
        out_type=jax.ShapeDtypeStruct((NC, NP, Dm), jnp.float32),
        mesh=_mesh,
        compiler_params=_sc_params,
        scratch_types=[
            pltpu.VMEM((NP,), jnp.float32),
            pltpu.VMEM((C,), jnp.int32),
            pltpu.VMEM((C,), jnp.int32),
            pltpu.VMEM((C,), jnp.float32),
            pltpu.VMEM((C, Dm), jnp.float32),
            pltpu.VMEM((SP, Dm), jnp.float32),
            pltpu.VMEM_SHARED((NP, Dm), jnp.float32),
            pltpu.SemaphoreType.DMA,
        ],
    )
    def msg(h_hbm, src_hbm, dst_hbm, ew_hbm, dinv_hbm, mp_hbm,
            dinv_v, idx_s, idx_d, ew_v, rows, zbuf, acc_sh, sem):
        cid = lax.axis_index("c")
        sid = lax.axis_index("s")
        wid = sid * NC + cid

        pltpu.sync_copy(dinv_hbm, dinv_v)
        _zero_vmem(zbuf, SP, Dm)
        pltpu.sync_copy(zbuf, acc_sh.at[pl.ds(sid * SP, SP)])
        plsc.subcore_barrier()

        @pl.loop(0, NCHUNK)
        def _(i):
            base = wid * ET + i * C
            pltpu.sync_copy(src_hbm.at[pl.ds(base, C)], idx_s)
            pltpu.sync_copy(dst_hbm.at[pl.ds(base, C)], idx_d)
            pltpu.sync_copy(ew_hbm.at[pl.ds(base, C)], ew_v)
            pltpu.async_copy(h_hbm.at[idx_s], rows, sem).wait()

            @pl.loop(0, C // L)
            def _(g):
                isv = idx_s[pl.ds(g * L, L)]
                idv = idx_d[pl.ds(g * L, L)]
                ds_ = plsc.load_gather(dinv_v, [isv])
                dd_ = plsc.load_gather(dinv_v, [idv])
                w = ds_ * ew_v[pl.ds(g * L, L)] * dd_
                for e in range(L):
                    row = g * L + e
                    wv = jnp.broadcast_to(w[e], (L,))
                    for k in range(Dm // L):
                        rows[row, pl.ds(k * L, L)] = (
                            rows[row, pl.ds(k * L, L)] * wv)

            pltpu.sync_copy(rows, acc_sh.at[idx_d], add=True)

        plsc.subcore_barrier()
        pltpu.sync_copy(acc_sh.at[pl.ds(sid * SP, SP)],
                        mp_hbm.at[cid, pl.ds(sid * SP, SP)])

    return msg(h, src, dst, ew, dinv)


def _tc_call(body, out_shape, *args):
    return pl.pallas_call(body, out_shape=out_shape)(*args)


def _prep1(x, W1):
    def body(x_ref, w_ref, xn_ref, h1_ref):
        xv = x_ref[...]
        s = jnp.sum(xv * xv, axis=1, keepdims=True)
        na = jnp.maximum(jnp.sqrt(s), 1e-8)
        xn_ref[...] = xv / na
        h1_ref[...] = jnp.dot(xv, w_ref[...],
                              preferred_element_type=jnp.float32)
    return _tc_call(
        body,
        (jax.ShapeDtypeStruct((N, x.shape[1]), jnp.float32),
         jax.ShapeDtypeStruct((N, W1.shape[1]), jnp.float32)),
        x, W1)


def _dinv_of(degp):
    def body(degp_ref, dinv_ref):
        deg = 1.0 + jnp.sum(degp_ref[...], axis=(0, 2), keepdims=True)
        dinv_ref[...] = lax.rsqrt(deg)
    return _tc_call(body, jax.ShapeDtypeStruct((1, NP, 1), jnp.float32), degp)


def _mid(mp, h1, dinv_col, b1_row, W2):
    def body(mp_ref, h1_ref, dc_ref, b_ref, w_ref, hn_ref, h2_ref):
        dc = dc_ref[...][:N]
        h = (mp_ref[0][:N] + mp_ref[1][:N]
             + dc * dc * h1_ref[...] + b_ref[...])
        h = jnp.maximum(h, 0.0)
        s = jnp.sum(h * h, axis=1, keepdims=True)
        na = jnp.maximum(jnp.sqrt(s), 1e-8)
        hn_ref[...] = h / na
        h2_ref[...] = jnp.dot(h, w_ref[...],
                              preferred_element_type=jnp.float32)
    return _tc_call(
        body,
        (jax.ShapeDtypeStruct((N, h1.shape[1]), jnp.float32),
         jax.ShapeDtypeStruct((N, W2.shape[1]), jnp.float32)),
        mp, h1, dinv_col, b1_row, W2)


def _final(mp, h2, dinv_col, b2_row):
    def body(mp_ref, h2_ref, dc_ref, b_ref, out_ref):
        dc = dc_ref[...][:N]
        out_ref[...] = (mp_ref[0][:N] + mp_ref[1][:N]
                        + dc * dc * h2_ref[...] + b_ref[...])
    return _tc_call(
        body, jax.ShapeDtypeStruct((N, h2.shape[1]), jnp.float32),
        mp, h2, dinv_col, b2_row)


def kernel(x, adj, W1, b1, W2, b2):
    src = adj[0].astype(jnp.int32)
    dst = adj[1].astype(jnp.int32)

    xn, h1 = _prep1(x, W1)
    ew1, degp1 = _att_pass(xn, src, dst, x.shape[1])
    dinv1_3 = _dinv_of(degp1)
    mp1 = _msg_pass(h1, src, dst, ew1, dinv1_3.reshape(NP), h1.shape[1])
    hn, h2 = _mid(mp1, h1, dinv1_3.reshape(NP, 1), b1.reshape(1, -1), W2)
    ew2, degp2 = _att_pass(hn, src, dst, hn.shape[1])
    dinv2_3 = _dinv_of(degp2)
    mp2 = _msg_pass(h2, src, dst, ew2, dinv2_3.reshape(NP), h2.shape[1])
    return _final(mp2, h2, dinv2_3.reshape(NP, 1), b2.reshape(1, -1))


# speedup vs baseline: 22.6592x; 2.3154x over previous
import dataclasses
import functools

import jax
import jax.numpy as jnp
from jax import lax
from jax.experimental import pallas as pl
from jax.experimental.pallas import tpu as pltpu
from jax.experimental.pallas import tpu_sc as plsc

N = 10000
E = 320000
NC = 2
NS = 16
NW = NC * NS
ET = E // NW
C = 80
NCHUNK = ET // C
L = 16
NP = 10112
SP = NP // NS

_mesh = plsc.VectorSubcoreMesh(core_axis_name="c", subcore_axis_name="s")

_sc_params = pltpu.CompilerParams()
if "needs_layout_passes" in pltpu.CompilerParams.__dataclass_fields__:
    _sc_params = dataclasses.replace(
        _sc_params, needs_layout_passes=False, use_tc_tiling_on_sc=False)


def _att_pass(xn, src3, dst3, zrows, D):
    @functools.partial(
        pl.kernel,
        out_type=(
            jax.ShapeDtypeStruct((NW, NCHUNK, C), jnp.float32),
            jax.ShapeDtypeStruct((NC, NP, L), jnp.float32),
        ),
        mesh=_mesh,
        compiler_params=_sc_params,
        scratch_types=[
            pltpu.VMEM((NCHUNK, C), jnp.int32),
            pltpu.VMEM((NCHUNK, C), jnp.int32),
            pltpu.VMEM((C, D), jnp.float32),
            pltpu.VMEM((C, D), jnp.float32),
            pltpu.VMEM((C, D), jnp.float32),
            pltpu.VMEM((C, D), jnp.float32),
            pltpu.VMEM((NCHUNK, C), jnp.float32),
            pltpu.VMEM((C, L), jnp.float32),
            pltpu.VMEM_SHARED((NP, L), jnp.float32),
            pltpu.SemaphoreType.DMA,
            pltpu.SemaphoreType.DMA,
            pltpu.SemaphoreType.DMA,
            pltpu.SemaphoreType.DMA,
        ],
    )
    def att(xn_hbm, src_hbm, dst_hbm, z_hbm, ew_hbm, degp_hbm,
            idx_sa, idx_da, a0, a1, b0, b1, ew_all, drow, deg_sh,
            sa0, sa1, sb0, sb1):
        cid = lax.axis_index("c")
        sid = lax.axis_index("s")
        wid = sid * NC + cid
        a_bufs, b_bufs = (a0, a1), (b0, b1)
        sas, sbs = (sa0, sa1), (sb0, sb1)

        pltpu.sync_copy(z_hbm.at[pl.ds(0, SP)], deg_sh.at[pl.ds(sid * SP, SP)])
        pltpu.sync_copy(z_hbm.at[pl.ds(0, C)], drow)
        pltpu.sync_copy(src_hbm.at[wid], idx_sa)
        pltpu.sync_copy(dst_hbm.at[wid], idx_da)
        plsc.subcore_barrier()

        lane_iota = lax.iota(jnp.int32, L)
        zeros_i = jnp.zeros((L,), jnp.int32)

        def issue(i, p):
            pltpu.async_copy(xn_hbm.at[idx_sa.at[i]], a_bufs[p], sas[p])
            pltpu.async_copy(xn_hbm.at[idx_da.at[i]], b_bufs[p], sbs[p])

        def step(i, p, issue_next):
            if issue_next:
                issue(i + 1, 1 - p)
            pltpu.make_async_copy(
                xn_hbm.at[idx_sa.at[i]], a_bufs[p], sas[p]).wait()
            pltpu.make_async_copy(
                xn_hbm.at[idx_da.at[i]], b_bufs[p], sbs[p]).wait()
            a_buf, b_buf = a_bufs[p], b_bufs[p]

            @pl.loop(0, C // L)
            def _(g):
                sims = jnp.zeros((L,), jnp.float32)
                for e in range(L):
                    row = g * L + e
                    acc = a_buf[row, pl.ds(0, L)] * b_buf[row, pl.ds(0, L)]
                    for k in range(1, D // L):
                        acc += (a_buf[row, pl.ds(k * L, L)]
                                * b_buf[row, pl.ds(k * L, L)])
                    sims = jnp.where(lane_iota == e,
                                     jnp.broadcast_to(jnp.sum(acc), (L,)),
                                     sims)
                sims = jnp.where(sims < 0.1, 0.0, sims)
                ew_all[i, pl.ds(g * L, L)] = sims
                plsc.store_scatter(drow, [g * L + lane_iota, zeros_i], sims)

            pltpu.sync_copy(drow, deg_sh.at[idx_da.at[i]], add=True)

        issue(0, 0)

        @pl.loop(0, (NCHUNK - 1) // 2)
        def _(j):
            step(2 * j, 0, True)
            step(2 * j + 1, 1, True)

        step(NCHUNK - 1, 0, False)

        pltpu.sync_copy(ew_all, ew_hbm.at[wid])
        plsc.subcore_barrier()
        pltpu.sync_copy(deg_sh.at[pl.ds(sid * SP, SP)],
                        degp_hbm.at[cid, pl.ds(sid * SP, SP)])

    return att(xn, src3, dst3, zrows)


def _msg_pass(h, src3, dst3, ew3, dinv, zrows, Dm):
    @functools.partial(
        pl.kernel,
        out_type=jax.ShapeDtypeStruct((NC, NP, Dm), jnp.float32),
        mesh=_mesh,
        compiler_params=_sc_params,
        scratch_types=[
            pltpu.VMEM((NP,), jnp.float32),
            pltpu.VMEM((NCHUNK, C), jnp.int32),
            pltpu.VMEM((NCHUNK, C), jnp.int32),
            pltpu.VMEM((NCHUNK, C), jnp.float32),
            pltpu.VMEM((C, Dm), jnp.float32),
            pltpu.VMEM((C, Dm), jnp.float32),
            pltpu.VMEM_SHARED((NP, Dm), jnp.float32),
            pltpu.SemaphoreType.DMA,
            pltpu.SemaphoreType.DMA,
        ],
    )
    def msg(h_hbm, src_hbm, dst_hbm, ew_hbm, dinv_hbm, z_hbm, mp_hbm,
            dinv_v, idx_sa, idx_da, ew_all, r0, r1, acc_sh, s0, s1):
        cid = lax.axis_index("c")
        sid = lax.axis_index("s")
        wid = sid * NC + cid
        rows_bufs, sems = (r0, r1), (s0, s1)

        pltpu.sync_copy(z_hbm, acc_sh.at[pl.ds(sid * SP, SP)])
        pltpu.sync_copy(dinv_hbm, dinv_v)
        pltpu.sync_copy(src_hbm.at[wid], idx_sa)
        pltpu.sync_copy(dst_hbm.at[wid], idx_da)
        pltpu.sync_copy(ew_hbm.at[wid], ew_all)
        plsc.subcore_barrier()

        def issue(i, p):
            pltpu.async_copy(h_hbm.at[idx_sa.at[i]], rows_bufs[p], sems[p])

        def step(i, p, issue_next):
            if issue_next:
                issue(i + 1, 1 - p)
            pltpu.make_async_copy(
                h_hbm.at[idx_sa.at[i]], rows_bufs[p], sems[p]).wait()
            rows = rows_bufs[p]

            @pl.loop(0, C // L)
            def _(g):
                isv = idx_sa[i, pl.ds(g * L, L)]
                idv = idx_da[i, pl.ds(g * L, L)]
                ds_ = plsc.load_gather(dinv_v, [isv])
                dd_ = plsc.load_gather(dinv_v, [idv])
                w = ds_ * ew_all[i, pl.ds(g * L, L)] * dd_
                for e in range(L):
                    row = g * L + e
                    wv = jnp.broadcast_to(w[e], (L,))
                    for k in range(Dm // L):
                        rows[row, pl.ds(k * L, L)] = (
                            rows[row, pl.ds(k * L, L)] * wv)

            pltpu.sync_copy(rows, acc_sh.at[idx_da.at[i]], add=True)

        issue(0, 0)

        @pl.loop(0, (NCHUNK - 1) // 2)
        def _(j):
            step(2 * j, 0, True)
            step(2 * j + 1, 1, True)

        step(NCHUNK - 1, 0, False)

        plsc.subcore_barrier()
        pltpu.sync_copy(acc_sh.at[pl.ds(sid * SP, SP)],
                        mp_hbm.at[cid, pl.ds(sid * SP, SP)])

    return msg(h, src3, dst3, ew3, dinv, zrows)


def _tc_call(body, out_shape, *args):
    return pl.pallas_call(body, out_shape=out_shape)(*args)


def _prep1(x, W1):
    def body(x_ref, w_ref, xn_ref, h1_ref):
        xv = x_ref[...]
        s = jnp.sum(xv * xv, axis=1, keepdims=True)
        na = jnp.maximum(jnp.sqrt(s), 1e-8)
        xn_ref[...] = xv / na
        h1_ref[...] = jnp.dot(xv, w_ref[...],
                              preferred_element_type=jnp.float32)
    return _tc_call(
        body,
        (jax.ShapeDtypeStruct((N, x.shape[1]), jnp.float32),
         jax.ShapeDtypeStruct((N, W1.shape[1]), jnp.float32)),
        x, W1)


def _dinv_of(degp):
    def body(degp_ref, dinv_ref):
        deg = 1.0 + jnp.sum(degp_ref[...], axis=(0, 2), keepdims=True)
        dinv_ref[...] = lax.rsqrt(deg)
    return _tc_call(body, jax.ShapeDtypeStruct((1, NP, 1), jnp.float32), degp)


def _mid(mp, h1, dinv_col, b1_row, W2):
    def body(mp_ref, h1_ref, dc_ref, b_ref, w_ref, hn_ref, h2_ref):
        dc = dc_ref[...][:N]
        h = (mp_ref[0][:N] + mp_ref[1][:N]
             + dc * dc * h1_ref[...] + b_ref[...])
        h = jnp.maximum(h, 0.0)
        s = jnp.sum(h * h, axis=1, keepdims=True)
        na = jnp.maximum(jnp.sqrt(s), 1e-8)
        hn_ref[...] = h / na
        h2_ref[...] = jnp.dot(h, w_ref[...],
                              preferred_element_type=jnp.float32)
    return _tc_call(
        body,
        (jax.ShapeDtypeStruct((N, h1.shape[1]), jnp.float32),
         jax.ShapeDtypeStruct((N, W2.shape[1]), jnp.float32)),
        mp, h1, dinv_col, b1_row, W2)


def _final(mp, h2, dinv_col, b2_row):
    def body(mp_ref, h2_ref, dc_ref, b_ref, out_ref):
        dc = dc_ref[...][:N]
        out_ref[...] = (mp_ref[0][:N] + mp_ref[1][:N]
                        + dc * dc * h2_ref[...] + b_ref[...])
    return _tc_call(
        body, jax.ShapeDtypeStruct((N, h2.shape[1]), jnp.float32),
        mp, h2, dinv_col, b2_row)


def kernel(x, adj, W1, b1, W2, b2):
    src3 = adj[0].astype(jnp.int32).reshape(NW, NCHUNK, C)
    dst3 = adj[1].astype(jnp.int32).reshape(NW, NCHUNK, C)
    z16 = jnp.zeros((SP, L), jnp.float32)

    xn, h1 = _prep1(x, W1)
    ew1, degp1 = _att_pass(xn, src3, dst3, z16, x.shape[1])
    dinv1_3 = _dinv_of(degp1)
    mp1 = _msg_pass(h1, src3, dst3, ew1, dinv1_3.reshape(NP),
                    jnp.zeros((SP, h1.shape[1]), jnp.float32), h1.shape[1])
    hn, h2 = _mid(mp1, h1, dinv1_3.reshape(NP, 1), b1.reshape(1, -1), W2)
    ew2, degp2 = _att_pass(hn, src3, dst3, z16, hn.shape[1])
    dinv2_3 = _dinv_of(degp2)
    mp2 = _msg_pass(h2, src3, dst3, ew2, dinv2_3.reshape(NP),
                    jnp.zeros((SP, h2.shape[1]), jnp.float32), h2.shape[1])
    return _final(mp2, h2, dinv2_3.reshape(NP, 1), b2.reshape(1, -1))
